# baseline (speedup 1.0000x reference)
import functools
import math

import jax
import jax.numpy as jnp
from jax import lax
from jax.experimental import pallas as pl
from jax.experimental.pallas import tpu as pltpu
from jax.experimental.pallas import tpu_sc as plsc

D_MODEL_ = 1024
D_FF_ = 4096
NE_ = 8
T_ = 2048
T_TILE = 256
MAX_TILES = 8
F_BLK = 1024
NF_ = D_FF_ // F_BLK
N_SLOTS = NE_ * MAX_TILES * T_TILE
DUMP_TILE = NE_ * MAX_TILES
NC_ = 2
NS_ = 16
NW_ = NC_ * NS_
TOK_W = T_ // NW_
CHUNK = 16


def _router_body(x_ref, wg_ref, slot1_ref, slot2_ref, p1_ref, p2_ref, nt_ref):
    x = x_ref[...]
    logits = jnp.dot(x, wg_ref[...], preferred_element_type=jnp.float32)
    col = lax.broadcasted_iota(jnp.int32, (T_, NE_), 1)
    m1 = jnp.max(logits, axis=1, keepdims=True)
    a1 = jnp.min(jnp.where(logits == m1, col, NE_), axis=1, keepdims=True)
    masked = jnp.where(col == a1, -jnp.inf, logits)
    m2 = jnp.max(masked, axis=1, keepdims=True)
    a2 = jnp.min(jnp.where(masked == m2, col, NE_), axis=1, keepdims=True)
    dd = jnp.exp(m2 - m1)
    p1_ref[...] = jnp.broadcast_to(1.0 / (1.0 + dd), (T_, 16))
    p2_ref[...] = jnp.broadcast_to(dd / (1.0 + dd), (T_, 16))

    oh1 = (col == a1).astype(jnp.float32)
    oh2 = (col == a2).astype(jnp.float32)
    c = oh1 + oh2
    ri = lax.broadcasted_iota(jnp.int32, (T_, T_), 0)
    ci = lax.broadcasted_iota(jnp.int32, (T_, T_), 1)
    tri = (ci < ri).astype(jnp.float32)
    pos = jnp.dot(tri, c, preferred_element_type=jnp.float32)
    counts = jnp.sum(c, axis=0, keepdims=True)
    nt = (counts.astype(jnp.int32) + T_TILE - 1) // T_TILE
    nt_ref[...] = nt
    dest = pos + (MAX_TILES * T_TILE) * col.astype(jnp.float32)
    slot1_ref[...] = jnp.sum(oh1 * dest, axis=1, keepdims=True).astype(jnp.int32)
    slot2_ref[...] = jnp.sum(oh2 * dest, axis=1, keepdims=True).astype(jnp.int32)


def _route(xf, Wg):
    return pl.pallas_call(
        _router_body,
        out_shape=[
            jax.ShapeDtypeStruct((T_, 1), jnp.int32),
            jax.ShapeDtypeStruct((T_, 1), jnp.int32),
            jax.ShapeDtypeStruct((T_, 16), jnp.float32),
            jax.ShapeDtypeStruct((T_, 16), jnp.float32),
            jax.ShapeDtypeStruct((1, NE_), jnp.int32),
        ],
    )(xf, Wg)


def _dispatch_body(x_hbm, s1_hbm, s2_hbm, xbuf_hbm, idx1_v, idx2_v, rows_v, sem):
    wid = lax.axis_index("s") * NC_ + lax.axis_index("c")
    base = pl.multiple_of(wid * TOK_W, TOK_W)
    pltpu.sync_copy(s1_hbm.at[pl.ds(base, TOK_W)], idx1_v)
    pltpu.sync_copy(s2_hbm.at[pl.ds(base, TOK_W)], idx2_v)
    pltpu.sync_copy(x_hbm.at[pl.ds(base, TOK_W)], rows_v)
    c1 = pltpu.async_copy(rows_v, xbuf_hbm.at[idx1_v], sem)
    c2 = pltpu.async_copy(rows_v, xbuf_hbm.at[idx2_v], sem)
    c1.wait()
    c2.wait()


RING = 5
N_PHASES = NE_ * NF_


def _issue_w(phase, w1_hbm, w2_hbm, w1r, w2r, sems):
    e2 = phase // NF_
    f2 = phase % NF_
    slot = lax.rem(phase, RING)
    pltpu.make_async_copy(
        w1_hbm.at[e2, :, pl.ds(f2 * F_BLK, F_BLK)], w1r.at[slot], sems.at[0, slot]
    ).start()
    pltpu.make_async_copy(
        w2_hbm.at[e2, pl.ds(f2 * F_BLK, F_BLK), :], w2r.at[slot], sems.at[1, slot]
    ).start()


def _ffn_body(nt_ref, x_ref, w1_hbm, b1_ref, w2_hbm, b2_ref, y_ref,
              acc_ref, w1r, w2r, sems):
    e = pl.program_id(0)
    f = pl.program_id(1)
    t = pl.program_id(2)
    p = e * NF_ + f
    slot = lax.rem(p, RING)

    @pl.when(t == 0)
    def _():
        @pl.when(p == 0)
        def _():
            for q in range(RING - 1):
                _issue_w(q, w1_hbm, w2_hbm, w1r, w2r, sems)

        @pl.when(p + RING - 1 < N_PHASES)
        def _():
            _issue_w(p + RING - 1, w1_hbm, w2_hbm, w1r, w2r, sems)

        pltpu.make_async_copy(w1_hbm.at[0, :, pl.ds(0, F_BLK)], w1r.at[slot],
                              sems.at[0, slot]).wait()
        pltpu.make_async_copy(w2_hbm.at[0, pl.ds(0, F_BLK), :], w2r.at[slot],
                              sems.at[1, slot]).wait()

    @pl.when(t < nt_ref[0, e])
    def _():
        x = x_ref[...]
        h = jnp.dot(x, w1r[slot], preferred_element_type=jnp.float32) + b1_ref[0]
        h = 0.5 * h * (1.0 + jax.lax.erf(h / math.sqrt(2.0)))
        contrib = jnp.dot(h, w2r[slot], preferred_element_type=jnp.float32)

        @pl.when(f == 0)
        def _():
            acc_ref[pl.ds(t * T_TILE, T_TILE), :] = contrib + b2_ref[0]

        @pl.when(f > 0)
        def _():
            acc_ref[pl.ds(t * T_TILE, T_TILE), :] += contrib

        @pl.when(f == NF_ - 1)
        def _():
            y_ref[...] = acc_ref[pl.ds(t * T_TILE, T_TILE), :]


def _last_active(nt_ref, e, t):
    return e * MAX_TILES + jnp.minimum(t, jnp.maximum(nt_ref[0, e] - 1, 0))


def _ffn(nt, xbuf, W1, b1r, W2, b2r):
    grid_spec = pltpu.PrefetchScalarGridSpec(
        num_scalar_prefetch=1,
        grid=(NE_, NF_, MAX_TILES),
        in_specs=[
            pl.BlockSpec((T_TILE, D_MODEL_),
                         lambda e, f, t, nt: (_last_active(nt, e, t), 0)),
            pl.BlockSpec(memory_space=pl.ANY),
            pl.BlockSpec((1, 1, F_BLK), lambda e, f, t, nt: (e, 0, f)),
            pl.BlockSpec(memory_space=pl.ANY),
            pl.BlockSpec((1, 1, D_MODEL_), lambda e, f, t, nt: (e, 0, 0)),
        ],
        out_specs=pl.BlockSpec(
            (T_TILE, D_MODEL_),
            lambda e, f, t, nt: (
                jnp.where((t < nt[0, e]) & (f == NF_ - 1),
                          e * MAX_TILES + t, DUMP_TILE),
                0,
            ),
        ),
        scratch_shapes=[
            pltpu.VMEM((MAX_TILES * T_TILE, D_MODEL_), jnp.float32),
            pltpu.VMEM((RING, D_MODEL_, F_BLK), jnp.float32),
            pltpu.VMEM((RING, F_BLK, D_MODEL_), jnp.float32),
            pltpu.SemaphoreType.DMA((2, RING)),
        ],
    )
    return pl.pallas_call(
        _ffn_body,
        grid_spec=grid_spec,
        out_shape=jax.ShapeDtypeStruct(((NE_ * MAX_TILES + 1) * T_TILE, D_MODEL_),
                                       jnp.float32),
        compiler_params=pltpu.CompilerParams(
            dimension_semantics=("arbitrary", "arbitrary", "arbitrary"),
        ),
    )(nt, xbuf, W1, b1r, W2, b2r)


def _combine_body(y_hbm, s1_hbm, s2_hbm, p1_hbm, p2_hbm, out_hbm,
                  idx1_v, idx2_v, p1_v, p2_v, r1_v, r2_v, sems):
    wid = lax.axis_index("s") * NC_ + lax.axis_index("c")
    base = pl.multiple_of(wid * TOK_W, TOK_W)
    pltpu.sync_copy(s1_hbm.at[pl.ds(base, TOK_W)], idx1_v)
    pltpu.sync_copy(s2_hbm.at[pl.ds(base, TOK_W)], idx2_v)
    pltpu.sync_copy(p1_hbm.at[pl.ds(base, TOK_W)], p1_v)
    pltpu.sync_copy(p2_hbm.at[pl.ds(base, TOK_W)], p2_v)

    nch = TOK_W // CHUNK

    def issue(c):
        par = c % 2
        iv1 = idx1_v[pl.ds(c * CHUNK, CHUNK)]
        iv2 = idx2_v[pl.ds(c * CHUNK, CHUNK)]
        d1 = pltpu.async_copy(y_hbm.at[iv1], r1_v.at[par], sems.at[par, 0])
        d2 = pltpu.async_copy(y_hbm.at[iv2], r2_v.at[par], sems.at[par, 1])
        return d1, d2

    pend = {0: issue(0)}
    for c in range(nch):
        par = c % 2
        if c + 1 < nch:
            pend[c + 1] = issue(c + 1)
        d1, d2 = pend.pop(c)
        d1.wait()
        d2.wait()

        @plsc.parallel_loop(0, CHUNK)
        def _(i, c=c, par=par):
            w1v = p1_v[c * CHUNK + i]
            w2v = p2_v[c * CHUNK + i]
            for j in range(D_MODEL_ // 16):
                sl = pl.ds(j * 16, 16)
                r1_v[par, i, sl] = (w1v * r1_v[par, i, sl]
                                    + w2v * r2_v[par, i, sl])

        pltpu.sync_copy(r1_v.at[par],
                        out_hbm.at[pl.ds(base + c * CHUNK, CHUNK)])


@functools.lru_cache(maxsize=None)
def _sc_kernels():
    mesh = plsc.VectorSubcoreMesh(core_axis_name="c", subcore_axis_name="s")
    dispatch = pl.kernel(
        _dispatch_body,
        out_type=jax.ShapeDtypeStruct((N_SLOTS, D_MODEL_), jnp.float32),
        mesh=mesh,
        scratch_types=[
            pltpu.VMEM((TOK_W,), jnp.int32),
            pltpu.VMEM((TOK_W,), jnp.int32),
            pltpu.VMEM((TOK_W, D_MODEL_), jnp.float32),
            pltpu.SemaphoreType.DMA,
        ],
    )
    combine = pl.kernel(
        _combine_body,
        out_type=jax.ShapeDtypeStruct((T_, D_MODEL_), jnp.float32),
        mesh=mesh,
        scratch_types=[
            pltpu.VMEM((TOK_W,), jnp.int32),
            pltpu.VMEM((TOK_W,), jnp.int32),
            pltpu.VMEM((TOK_W, 16), jnp.float32),
            pltpu.VMEM((TOK_W, 16), jnp.float32),
            pltpu.VMEM((2, CHUNK, D_MODEL_), jnp.float32),
            pltpu.VMEM((2, CHUNK, D_MODEL_), jnp.float32),
            pltpu.SemaphoreType.DMA((2, 2)),
        ],
    )
    return dispatch, combine

def kernel(x, Wg, W1, b1, W2, b2):
    B, S, d = x.shape
    xf = x.reshape(-1, d)
    slot1, slot2, p1, p2, nt = _route(xf, Wg)
    slot1 = slot1.reshape(T_)
    slot2 = slot2.reshape(T_)
    dispatch, combine = _sc_kernels()
    xbuf = dispatch(xf, slot1, slot2)
    y = _ffn(nt, xbuf, W1, b1.reshape(NE_, 1, D_FF_), W2,
             b2.reshape(NE_, 1, D_MODEL_))
    out = combine(y, slot1, slot2, p1, p2)
    return out.reshape(B, S, d)


# speedup vs baseline: 1.0130x; 1.0130x over previous
import functools
import math

import jax
import jax.numpy as jnp
from jax import lax
from jax.experimental import pallas as pl
from jax.experimental.pallas import tpu as pltpu
from jax.experimental.pallas import tpu_sc as plsc

D_MODEL_ = 1024
D_FF_ = 4096
NE_ = 8
T_ = 2048
T_TILE = 256
MAX_TILES = 8
F_BLK = 1024
NF_ = D_FF_ // F_BLK
N_SLOTS = NE_ * MAX_TILES * T_TILE
DUMP_TILE = NE_ * MAX_TILES
NC_ = 2
NS_ = 16
NW_ = NC_ * NS_
TOK_W = T_ // NW_
CHUNK = 16


def _router_body(x_ref, wg_ref, slot1_ref, slot2_ref, p1_ref, p2_ref, nt_ref):
    x = x_ref[...]
    logits = jnp.dot(x, wg_ref[...], preferred_element_type=jnp.float32)
    col = lax.broadcasted_iota(jnp.int32, (T_, NE_), 1)
    m1 = jnp.max(logits, axis=1, keepdims=True)
    a1 = jnp.min(jnp.where(logits == m1, col, NE_), axis=1, keepdims=True)
    masked = jnp.where(col == a1, -jnp.inf, logits)
    m2 = jnp.max(masked, axis=1, keepdims=True)
    a2 = jnp.min(jnp.where(masked == m2, col, NE_), axis=1, keepdims=True)
    dd = jnp.exp(m2 - m1)
    p1_ref[...] = jnp.broadcast_to(1.0 / (1.0 + dd), (T_, 16))
    p2_ref[...] = jnp.broadcast_to(dd / (1.0 + dd), (T_, 16))

    oh1 = (col == a1).astype(jnp.float32)
    oh2 = (col == a2).astype(jnp.float32)
    c = oh1 + oh2
    ri = lax.broadcasted_iota(jnp.int32, (T_, T_), 0)
    ci = lax.broadcasted_iota(jnp.int32, (T_, T_), 1)
    tri = (ci < ri).astype(jnp.float32)
    pos = jnp.dot(tri, c, preferred_element_type=jnp.float32)
    counts = jnp.sum(c, axis=0, keepdims=True)
    nt = (counts.astype(jnp.int32) + T_TILE - 1) // T_TILE
    nt_ref[...] = nt
    dest = pos + (MAX_TILES * T_TILE) * col.astype(jnp.float32)
    slot1_ref[...] = jnp.sum(oh1 * dest, axis=1, keepdims=True).astype(jnp.int32)
    slot2_ref[...] = jnp.sum(oh2 * dest, axis=1, keepdims=True).astype(jnp.int32)


def _route(xf, Wg):
    return pl.pallas_call(
        _router_body,
        out_shape=[
            jax.ShapeDtypeStruct((T_, 1), jnp.int32),
            jax.ShapeDtypeStruct((T_, 1), jnp.int32),
            jax.ShapeDtypeStruct((T_, 16), jnp.float32),
            jax.ShapeDtypeStruct((T_, 16), jnp.float32),
            jax.ShapeDtypeStruct((1, NE_), jnp.int32),
        ],
    )(xf, Wg)


def _dispatch_body(x_hbm, s1_hbm, s2_hbm, xbuf_hbm, idx1_v, idx2_v, rows_v, sem):
    wid = lax.axis_index("s") * NC_ + lax.axis_index("c")
    base = pl.multiple_of(wid * TOK_W, TOK_W)
    pltpu.sync_copy(s1_hbm.at[pl.ds(base, TOK_W)], idx1_v)
    pltpu.sync_copy(s2_hbm.at[pl.ds(base, TOK_W)], idx2_v)
    pltpu.sync_copy(x_hbm.at[pl.ds(base, TOK_W)], rows_v)
    c1 = pltpu.async_copy(rows_v, xbuf_hbm.at[idx1_v], sem)
    c2 = pltpu.async_copy(rows_v, xbuf_hbm.at[idx2_v], sem)
    c1.wait()
    c2.wait()


RING = 4
N_PHASES = NE_ * NF_


def _issue_w(phase, w1_hbm, w2_hbm, w1r, w2r, sems):
    e2 = phase // NF_
    f2 = phase % NF_
    slot = lax.rem(phase, RING)
    pltpu.make_async_copy(
        w1_hbm.at[e2, :, pl.ds(f2 * F_BLK, F_BLK)], w1r.at[slot], sems.at[0, slot]
    ).start()
    pltpu.make_async_copy(
        w2_hbm.at[e2, pl.ds(f2 * F_BLK, F_BLK), :], w2r.at[slot], sems.at[1, slot]
    ).start()


def _ffn_body(nt_ref, x_ref, w1_hbm, b1_ref, w2_hbm, b2_ref, y_ref,
              acc_ref, w1r, w2r, sems):
    e = pl.program_id(0)
    f = pl.program_id(1)
    t = pl.program_id(2)
    p = e * NF_ + f
    slot = lax.rem(p, RING)

    @pl.when(t == 0)
    def _():
        @pl.when(p == 0)
        def _():
            for q in range(RING - 1):
                _issue_w(q, w1_hbm, w2_hbm, w1r, w2r, sems)

        @pl.when(p + RING - 1 < N_PHASES)
        def _():
            _issue_w(p + RING - 1, w1_hbm, w2_hbm, w1r, w2r, sems)

        pltpu.make_async_copy(w1_hbm.at[0, :, pl.ds(0, F_BLK)], w1r.at[slot],
                              sems.at[0, slot]).wait()
        pltpu.make_async_copy(w2_hbm.at[0, pl.ds(0, F_BLK), :], w2r.at[slot],
                              sems.at[1, slot]).wait()

    @pl.when(t < nt_ref[0, e])
    def _():
        x = x_ref[...]
        h = jnp.dot(x, w1r[slot], preferred_element_type=jnp.float32) + b1_ref[0]
        h = 0.5 * h * (1.0 + jax.lax.erf(h / math.sqrt(2.0)))
        contrib = jnp.dot(h, w2r[slot], preferred_element_type=jnp.float32)

        @pl.when(f == 0)
        def _():
            acc_ref[pl.ds(t * T_TILE, T_TILE), :] = contrib + b2_ref[0]

        @pl.when(f > 0)
        def _():
            acc_ref[pl.ds(t * T_TILE, T_TILE), :] += contrib

        @pl.when(f == NF_ - 1)
        def _():
            y_ref[...] = acc_ref[pl.ds(t * T_TILE, T_TILE), :]


def _last_active(nt_ref, e, t):
    return e * MAX_TILES + jnp.minimum(t, jnp.maximum(nt_ref[0, e] - 1, 0))


def _ffn(nt, xbuf, W1, b1r, W2, b2r):
    grid_spec = pltpu.PrefetchScalarGridSpec(
        num_scalar_prefetch=1,
        grid=(NE_, NF_, MAX_TILES),
        in_specs=[
            pl.BlockSpec((T_TILE, D_MODEL_),
                         lambda e, f, t, nt: (_last_active(nt, e, t), 0)),
            pl.BlockSpec(memory_space=pl.ANY),
            pl.BlockSpec((1, 1, F_BLK), lambda e, f, t, nt: (e, 0, f)),
            pl.BlockSpec(memory_space=pl.ANY),
            pl.BlockSpec((1, 1, D_MODEL_), lambda e, f, t, nt: (e, 0, 0)),
        ],
        out_specs=pl.BlockSpec(
            (T_TILE, D_MODEL_),
            lambda e, f, t, nt: (
                jnp.where((t < nt[0, e]) & (f == NF_ - 1),
                          e * MAX_TILES + t, DUMP_TILE),
                0,
            ),
        ),
        scratch_shapes=[
            pltpu.VMEM((MAX_TILES * T_TILE, D_MODEL_), jnp.float32),
            pltpu.VMEM((RING, D_MODEL_, F_BLK), jnp.float32),
            pltpu.VMEM((RING, F_BLK, D_MODEL_), jnp.float32),
            pltpu.SemaphoreType.DMA((2, RING)),
        ],
    )
    return pl.pallas_call(
        _ffn_body,
        grid_spec=grid_spec,
        out_shape=jax.ShapeDtypeStruct(((NE_ * MAX_TILES + 1) * T_TILE, D_MODEL_),
                                       jnp.float32),
        compiler_params=pltpu.CompilerParams(
            dimension_semantics=("arbitrary", "arbitrary", "arbitrary"),
        ),
    )(nt, xbuf, W1, b1r, W2, b2r)


def _combine_body(y_hbm, s1_hbm, s2_hbm, p1_hbm, p2_hbm, out_hbm,
                  idx1_v, idx2_v, p1_v, p2_v, r1_v, r2_v, sems):
    wid = lax.axis_index("s") * NC_ + lax.axis_index("c")
    base = pl.multiple_of(wid * TOK_W, TOK_W)
    pltpu.sync_copy(s1_hbm.at[pl.ds(base, TOK_W)], idx1_v)
    pltpu.sync_copy(s2_hbm.at[pl.ds(base, TOK_W)], idx2_v)
    pltpu.sync_copy(p1_hbm.at[pl.ds(base, TOK_W)], p1_v)
    pltpu.sync_copy(p2_hbm.at[pl.ds(base, TOK_W)], p2_v)

    nch = TOK_W // CHUNK

    def issue(c):
        par = c % 2
        iv1 = idx1_v[pl.ds(c * CHUNK, CHUNK)]
        iv2 = idx2_v[pl.ds(c * CHUNK, CHUNK)]
        d1 = pltpu.async_copy(y_hbm.at[iv1], r1_v.at[par], sems.at[par, 0])
        d2 = pltpu.async_copy(y_hbm.at[iv2], r2_v.at[par], sems.at[par, 1])
        return d1, d2

    pend = {0: issue(0)}
    for c in range(nch):
        par = c % 2
        if c + 1 < nch:
            pend[c + 1] = issue(c + 1)
        d1, d2 = pend.pop(c)
        d1.wait()
        d2.wait()

        @plsc.parallel_loop(0, CHUNK)
        def _(i, c=c, par=par):
            w1v = p1_v[c * CHUNK + i]
            w2v = p2_v[c * CHUNK + i]
            for j in range(D_MODEL_ // 16):
                sl = pl.ds(j * 16, 16)
                r1_v[par, i, sl] = (w1v * r1_v[par, i, sl]
                                    + w2v * r2_v[par, i, sl])

        pltpu.sync_copy(r1_v.at[par],
                        out_hbm.at[pl.ds(base + c * CHUNK, CHUNK)])


@functools.lru_cache(maxsize=None)
def _sc_kernels():
    mesh = plsc.VectorSubcoreMesh(core_axis_name="c", subcore_axis_name="s")
    dispatch = pl.kernel(
        _dispatch_body,
        out_type=jax.ShapeDtypeStruct((N_SLOTS, D_MODEL_), jnp.float32),
        mesh=mesh,
        scratch_types=[
            pltpu.VMEM((TOK_W,), jnp.int32),
            pltpu.VMEM((TOK_W,), jnp.int32),
            pltpu.VMEM((TOK_W, D_MODEL_), jnp.float32),
            pltpu.SemaphoreType.DMA,
        ],
    )
    combine = pl.kernel(
        _combine_body,
        out_type=jax.ShapeDtypeStruct((T_, D_MODEL_), jnp.float32),
        mesh=mesh,
        scratch_types=[
            pltpu.VMEM((TOK_W,), jnp.int32),
            pltpu.VMEM((TOK_W,), jnp.int32),
            pltpu.VMEM((TOK_W, 16), jnp.float32),
            pltpu.VMEM((TOK_W, 16), jnp.float32),
            pltpu.VMEM((2, CHUNK, D_MODEL_), jnp.float32),
            pltpu.VMEM((2, CHUNK, D_MODEL_), jnp.float32),
            pltpu.SemaphoreType.DMA((2, 2)),
        ],
    )
    return dispatch, combine

def kernel(x, Wg, W1, b1, W2, b2):
    B, S, d = x.shape
    xf = x.reshape(-1, d)
    slot1, slot2, p1, p2, nt = _route(xf, Wg)
    slot1 = slot1.reshape(T_)
    slot2 = slot2.reshape(T_)
    dispatch, combine = _sc_kernels()
    xbuf = dispatch(xf, slot1, slot2)
    y = _ffn(nt, xbuf, W1, b1.reshape(NE_, 1, D_FF_), W2,
             b2.reshape(NE_, 1, D_MODEL_))
    out = combine(y, slot1, slot2, p1, p2)
    return out.reshape(B, S, d)
